# baseline (speedup 1.0000x reference)
import jax
import jax.numpy as jnp
from jax.experimental import pallas as pl
from jax.experimental.pallas import tpu as pltpu

_B, _T, _C = 128, 50, 8192
_ROWS = _B * _T
_BLK = 800
_NBLK = _ROWS // _BLK


def _onehot_body(prob_ref, lab_ref, out_ref):
    out_ref[...] = jnp.zeros((_BLK, _C), jnp.float32)


def kernel(LLM_label, prob):
    flat = LLM_label.reshape(_NBLK, 1, _BLK).astype(jnp.int32)
    prob2 = prob.reshape(1, 1)
    out = pl.pallas_call(
        _onehot_body,
        grid=(_NBLK,),
        in_specs=[
            pl.BlockSpec(memory_space=pltpu.SMEM),
            pl.BlockSpec((1, 1, _BLK), lambda i: (i, 0, 0)),
        ],
        out_specs=pl.BlockSpec((_BLK, _C), lambda i: (i, 0)),
        out_shape=jax.ShapeDtypeStruct((_ROWS, _C), jnp.float32),
    )(prob2, flat)
    return out.reshape(_B, _T, _C)


# speedup vs baseline: 1.0022x; 1.0022x over previous
import jax
import jax.numpy as jnp
from jax.experimental import pallas as pl
from jax.experimental.pallas import tpu as pltpu

_B, _T, _C = 128, 50, 8192
_ROWS = _B * _T
_CH = 128
_NCH = _ROWS // _CH
_NBUF = 8


def _onehot_body(prob_ref, lab_ref, out_ref, zbuf, sems):
    p = prob_ref[0, 0]

    def chunk_copy(c, b):
        return pltpu.make_async_copy(
            zbuf.at[b], out_ref.at[pl.ds(c * _CH, _CH), :], sems.at[b])

    for c in range(_NCH):
        b = c % _NBUF
        if c >= _NBUF:
            chunk_copy(c - _NBUF, b).wait()
        labs = lab_ref[c, 0, :]
        col = jax.lax.broadcasted_iota(jnp.int32, (_CH, _C), 1)
        zbuf[b] = jnp.where(col == labs[:, None], p, 0.0)
        chunk_copy(c, b).start()
    for c in range(max(_NCH - _NBUF, 0), _NCH):
        chunk_copy(c, c % _NBUF).wait()


def kernel(LLM_label, prob):
    flat = LLM_label.reshape(_NCH, 1, _CH).astype(jnp.int32)
    prob2 = prob.reshape(1, 1)
    out = pl.pallas_call(
        _onehot_body,
        in_specs=[
            pl.BlockSpec(memory_space=pltpu.MemorySpace.SMEM),
            pl.BlockSpec(memory_space=pltpu.MemorySpace.VMEM),
        ],
        out_specs=pl.BlockSpec(memory_space=pltpu.MemorySpace.HBM),
        out_shape=jax.ShapeDtypeStruct((_ROWS, _C), jnp.float32),
        scratch_shapes=[
            pltpu.VMEM((_NBUF, _CH, _C), jnp.float32),
            pltpu.SemaphoreType.DMA((_NBUF,)),
        ],
    )(prob2, flat)
    return out.reshape(_B, _T, _C)


# speedup vs baseline: 2.0442x; 2.0398x over previous
import jax
import jax.numpy as jnp
from jax.experimental import pallas as pl
from jax.experimental.pallas import tpu as pltpu

_B, _T, _C = 128, 50, 8192
_BB = 8
_NBLK = _B // _BB
_BLK = _BB * _T


def _onehot_body(prob_ref, lab_ref, out_ref):
    labs = lab_ref[0, 0, :]
    col = jax.lax.broadcasted_iota(jnp.int32, (_BLK, _C), 1)
    mask = col == labs[:, None]
    out_ref[...] = jnp.where(mask, prob_ref[0, 0], 0.0).reshape(_BB, _T, _C)


def kernel(LLM_label, prob):
    flat = LLM_label.reshape(_NBLK, 1, _BLK).astype(jnp.int32)
    prob2 = prob.reshape(1, 1)
    return pl.pallas_call(
        _onehot_body,
        grid=(_NBLK,),
        in_specs=[
            pl.BlockSpec(memory_space=pltpu.MemorySpace.SMEM),
            pl.BlockSpec((1, 1, _BLK), lambda i: (i, 0, 0)),
        ],
        out_specs=pl.BlockSpec((_BB, _T, _C), lambda i: (i, 0, 0)),
        out_shape=jax.ShapeDtypeStruct((_B, _T, _C), jnp.float32),
    )(prob2, flat)


# speedup vs baseline: 2.0449x; 1.0004x over previous
import jax
import jax.numpy as jnp
from jax.experimental import pallas as pl
from jax.experimental.pallas import tpu as pltpu

_B, _T, _C = 128, 50, 8192
_BB = 2
_NCH = _B // _BB
_RPC = _BB * _T
_NBUF = 8


def _onehot_body(prob_ref, lab_ref, out_ref, zbuf, sems):
    p = prob_ref[0, 0]

    def chunk_copy(c, b):
        return pltpu.make_async_copy(
            zbuf.at[b], out_ref.at[pl.ds(c * _BB, _BB), :, :], sems.at[b])

    for c in range(_NCH):
        b = c % _NBUF
        if c >= _NBUF:
            chunk_copy(c - _NBUF, b).wait()
        labs = lab_ref[c, 0, :]
        col = jax.lax.broadcasted_iota(jnp.int32, (_RPC, _C), 1)
        zbuf[b] = jnp.where(col == labs[:, None], p, 0.0).reshape(_BB, _T, _C)
        chunk_copy(c, b).start()
    for c in range(max(_NCH - _NBUF, 0), _NCH):
        chunk_copy(c, c % _NBUF).wait()


def kernel(LLM_label, prob):
    flat = LLM_label.reshape(_NCH, 1, _RPC).astype(jnp.int32)
    prob2 = prob.reshape(1, 1)
    return pl.pallas_call(
        _onehot_body,
        in_specs=[
            pl.BlockSpec(memory_space=pltpu.MemorySpace.SMEM),
            pl.BlockSpec(memory_space=pltpu.MemorySpace.VMEM),
        ],
        out_specs=pl.BlockSpec(memory_space=pltpu.MemorySpace.HBM),
        out_shape=jax.ShapeDtypeStruct((_B, _T, _C), jnp.float32),
        scratch_shapes=[
            pltpu.VMEM((_NBUF, _BB, _T, _C), jnp.float32),
            pltpu.SemaphoreType.DMA((_NBUF,)),
        ],
    )(prob2, flat)
